# baseline (speedup 1.0000x reference)
import jax
import jax.numpy as jnp
from jax import lax
from jax.experimental import pallas as pl
from jax.experimental.pallas import tpu as pltpu
from jax.experimental.pallas import tpu_sc as plsc

_NC = 2
_NS = 16
_NW = _NC * _NS
_C = 128
_KB = 16


def _sc_agg_body(x_hbm, srcr, dstr, z_row, agg_out,
                 idx_s, idx_d, rows0, rows1, acc_sh, semg0, semg1):
  ko_n = srcr.shape[1]
  cid = lax.axis_index("c")
  sid = lax.axis_index("s")
  wid = sid * _NC + cid
  nt = acc_sh.shape[0] // _NS
  pltpu.sync_copy(z_row, acc_sh.at[pl.ds(sid * nt, nt)])
  plsc.subcore_barrier()
  half = _KB // 2

  def outer(ko, carry):
    pltpu.sync_copy(srcr.at[wid, ko], idx_s)
    pltpu.sync_copy(dstr.at[wid, ko], idx_d)
    pltpu.async_copy(x_hbm.at[idx_s.at[0]], rows0, semg0)

    def inner(j2, c2):
      e = 2 * j2
      pltpu.async_copy(x_hbm.at[idx_s.at[e + 1]], rows1, semg1)
      pltpu.make_async_copy(x_hbm.at[idx_s.at[e]], rows0, semg0).wait()
      pltpu.sync_copy(rows0, acc_sh.at[idx_d.at[e]], add=True)

      @pl.when(j2 + 1 < half)
      def _():
        pltpu.async_copy(x_hbm.at[idx_s.at[e + 2]], rows0, semg0)

      pltpu.make_async_copy(x_hbm.at[idx_s.at[e + 1]], rows1, semg1).wait()
      pltpu.sync_copy(rows1, acc_sh.at[idx_d.at[e + 1]], add=True)
      return c2

    return lax.fori_loop(0, half, inner, carry)

  lax.fori_loop(0, ko_n, outer, 0)
  plsc.subcore_barrier()
  pltpu.sync_copy(acc_sh.at[pl.ds(sid * nt, nt)],
                  agg_out.at[cid, pl.ds(sid * nt, nt)])


def _make_sc_agg(nacc, D):
  mesh = plsc.VectorSubcoreMesh(core_axis_name="c", subcore_axis_name="s")
  return pl.kernel(
      _sc_agg_body,
      mesh=mesh,
      out_type=jax.ShapeDtypeStruct((_NC, nacc, D), jnp.float32),
      scratch_types=[
          pltpu.VMEM((_KB, _C), jnp.int32),
          pltpu.VMEM((_KB, _C), jnp.int32),
          pltpu.VMEM((_C, D), jnp.float32),
          pltpu.VMEM((_C, D), jnp.float32),
          pltpu.VMEM_SHARED((nacc, D), jnp.float32),
          pltpu.SemaphoreType.DMA,
          pltpu.SemaphoreType.DMA,
      ],
  )


def _sc_deg_body(dstr, z_hist, deg_out, idx_d, hist):
  ko_n = dstr.shape[1]
  cid = lax.axis_index("c")
  sid = lax.axis_index("s")
  wid = sid * _NC + cid
  nw = hist.shape[0]
  half_nodes = nw // 16
  lane = lax.iota(jnp.int32, 16)
  ones_f = jnp.full((16,), 1.0, jnp.float32)
  for hpass in range(2):
    base = hpass * half_nodes
    pltpu.sync_copy(z_hist, hist)

    def outer(ko, c, _base=base):
      pltpu.sync_copy(dstr.at[wid, ko], idx_d)

      def jloop(j, c2):
        for v in range(8):
          iv = idx_d[j, pl.ds(v * 16, 16)]
          lc = iv - _base
          m = (lc >= 0) & (lc < half_nodes)
          lcs = jnp.where(m, lc, 0)
          flat = jnp.left_shift(lcs, 4) + lane
          plsc.addupdate_scatter(hist, [flat], ones_f, mask=m)
        return c2

      return lax.fori_loop(0, _KB, jloop, c)

    lax.fori_loop(0, ko_n, outer, 0)
    pltpu.sync_copy(hist, deg_out.at[cid, sid, hpass])


def _make_sc_deg(nh):
  mesh = plsc.VectorSubcoreMesh(core_axis_name="c", subcore_axis_name="s")
  nw = nh * 8 * 16
  return pl.kernel(
      _sc_deg_body,
      mesh=mesh,
      out_type=jax.ShapeDtypeStruct((_NC, _NS, 2, nw), jnp.float32),
      scratch_types=[
          pltpu.VMEM((_KB, _C), jnp.int32),
          pltpu.VMEM((nw,), jnp.float32),
      ],
      compiler_params=pltpu.CompilerParams(needs_layout_passes=False),
  )


def _dotT(a, w):
  return lax.dot_general(a, w, (((1,), (1,)), ((), ())),
                         preferred_element_type=jnp.float32)


def _tcd_body(degp, degr_out):
  degr_out[...] = jnp.sum(degp[...], axis=(0, 1))


def _tcd(degp, nodes):
  R = 1000
  return pl.pallas_call(
      _tcd_body,
      grid=(nodes // R,),
      in_specs=[pl.BlockSpec((_NC, _NS, R, 16), lambda i: (0, 0, i, 0))],
      out_specs=pl.BlockSpec((R, 16), lambda i: (i, 0)),
      out_shape=jax.ShapeDtypeStruct((nodes, 16), jnp.float32),
  )(degp)


def _deg_from_hist(degr_ref):
  return jnp.sum(degr_ref[...], axis=1, keepdims=True)


def _tc1_body(aggp, degp, x, w1l, b1l, w1r, h_out):
  agg = aggp[0] + aggp[1]
  deg = _deg_from_hist(degp)
  a = agg / jnp.maximum(deg, 1.0)
  y = _dotT(a, w1l[...]) + b1l[...] + _dotT(x[...], w1r[...])
  h_out[...] = jnp.maximum(y, 0.0)


def _tc0_body(img, attr, rel, wi, bi, wa, ba, wr, br,
              im_out, at_out, re_out):
  im_out[...] = _dotT(img[...], wi[...]) + bi[...]
  at_out[...] = _dotT(attr[...], wa[...]) + ba[...]
  re_out[...] = _dotT(rel[...], wr[...]) + br[...]


def _tc2_body(aggp, degp, h, im_in, at_in, re_in,
              w2l, b2l, w2r, wg, bg, wf, bf,
              fused_out, h2_out):
  agg = aggp[0] + aggp[1]
  deg = _deg_from_hist(degp)
  a = agg / jnp.maximum(deg, 1.0)
  h2 = _dotT(a, w2l[...]) + b2l[...] + _dotT(h[...], w2r[...])
  g = _dotT(h2, wg[...]) + bg[...]
  im = im_in[...]
  at = at_in[...]
  re = re_in[...]

  wfv = wf[...]
  b = bf[:, 0:1]
  lg = jnp.sum(g * wfv, axis=1, keepdims=True) + b
  li = jnp.sum(im * wfv, axis=1, keepdims=True) + b
  la = jnp.sum(at * wfv, axis=1, keepdims=True) + b
  lr = jnp.sum(re * wfv, axis=1, keepdims=True) + b
  m = jnp.maximum(jnp.maximum(lg, li), jnp.maximum(la, lr))
  eg = jnp.exp(lg - m)
  ei = jnp.exp(li - m)
  ea = jnp.exp(la - m)
  er = jnp.exp(lr - m)
  s = eg + ei + ea + er
  fused = (eg * g + ei * im + ea * at + er * re) / s
  nrm = jnp.sqrt(jnp.sum(fused * fused, axis=1, keepdims=True))
  fused_out[...] = fused / jnp.maximum(nrm, 1e-12)
  h2_out[...] = h2


def _row_spec(R, D):
  return pl.BlockSpec((R, D), lambda i: (i, 0))


def _full_spec(shape):
  nd = len(shape)
  return pl.BlockSpec(shape, lambda i: (0,) * nd)


def _tc0(img, attr, rel, *ws):
  N, D = img.shape
  R = 2000
  out = jax.ShapeDtypeStruct((N, D), jnp.float32)
  w_specs = [
      _full_spec((D, D)), _full_spec((1, D)),
      _full_spec((D, D)), _full_spec((1, D)),
      _full_spec((D, D)), _full_spec((1, D)),
  ]
  return pl.pallas_call(
      _tc0_body,
      grid=(N // R,),
      in_specs=[_row_spec(R, D)] * 3 + w_specs,
      out_specs=[_row_spec(R, D)] * 3,
      out_shape=[out, out, out],
  )(img, attr, rel, *ws)


def _tc1(aggp, degp, x, w1l, b1l, w1r):
  N, D = x.shape
  R = 2000
  return pl.pallas_call(
      _tc1_body,
      grid=(N // R,),
      in_specs=[
          pl.BlockSpec((_NC, R, D), lambda i: (0, i, 0)),
          pl.BlockSpec((R, 16), lambda i: (i, 0)),
          _row_spec(R, D),
          _full_spec((D, D)),
          _full_spec((1, D)),
          _full_spec((D, D)),
      ],
      out_specs=_row_spec(R, D),
      out_shape=jax.ShapeDtypeStruct((N, D), jnp.float32),
  )(aggp, degp, x, w1l, b1l, w1r)


def _tc2(aggp, degp, h, im, at, re, *ws):
  N, D = h.shape
  R = 2000
  out = jax.ShapeDtypeStruct((N, D), jnp.float32)
  w_specs = [
      _full_spec((D, D)), _full_spec((1, D)), _full_spec((D, D)),
      _full_spec((D, D)), _full_spec((1, D)),
      _full_spec((1, D)), _full_spec((1, D)),
  ]
  return pl.pallas_call(
      _tc2_body,
      grid=(N // R,),
      in_specs=[
          pl.BlockSpec((_NC, R, D), lambda i: (0, i, 0)),
          pl.BlockSpec((R, 16), lambda i: (i, 0)),
          _row_spec(R, D), _row_spec(R, D), _row_spec(R, D), _row_spec(R, D),
      ] + w_specs,
      out_specs=[_row_spec(R, D)] * 2,
      out_shape=[out, out],
  )(aggp, degp, h, im, at, re, *ws)


def kernel(x, edge_index, img_emb, attr_emb, rel_emb,
           W1l, b1l, W1r, W2l, b2l, W2r, Wgph, bgph, Wimg, bimg,
           Watt, batt, Wrel, brel, Wfus, bfus):
  N, D = x.shape
  E = edge_index.shape[1]

  ko_n = -(-E // (_NW * _KB * _C))
  pad = _NW * ko_n * _KB * _C - E
  src = edge_index[0].astype(jnp.int32)
  dst = edge_index[1].astype(jnp.int32)
  nacc = ((N + 96 + 8 * _NS - 1) // (8 * _NS)) * (8 * _NS)
  if pad:
    ar = jnp.arange(pad, dtype=jnp.int32)
    src = jnp.concatenate([src, (ar * 97) % N])
    dst = jnp.concatenate([dst, N + (ar % 96)])
  srcr = src.reshape(_NW, ko_n, _KB, _C)
  dstr = dst.reshape(_NW, ko_n, _KB, _C)

  nt = nacc // _NS
  z_row = jnp.zeros((nt, D), jnp.float32)

  sc_agg = _make_sc_agg(nacc, D)
  im, at, re = _tc0(img_emb, attr_emb, rel_emb,
                    Wimg, bimg.reshape(1, D),
                    Watt, batt.reshape(1, D),
                    Wrel, brel.reshape(1, D))
  nh = -(-N // 16)
  z_hist = jnp.zeros((nh * 8 * 16,), jnp.float32)
  degw = _make_sc_deg(nh)(dstr, z_hist)
  degp = _tcd(degw.reshape(_NC, _NS, 2 * nh * 8, 16), 2 * nh * 8)
  agg1p = sc_agg(x, srcr, dstr, z_row)
  h = _tc1(agg1p, degp, x, W1l, b1l.reshape(1, D), W1r)
  agg2p = sc_agg(h, srcr, dstr, z_row)
  fused, h2 = _tc2(
      agg2p, degp, h, im, at, re,
      W2l, b2l.reshape(1, D), W2r,
      Wgph, bgph.reshape(1, D),
      Wfus, jnp.broadcast_to(bfus.reshape(1, 1), (1, D)))
  return fused, h2, im, at, re


# speedup vs baseline: 1.2633x; 1.2633x over previous
import jax
import jax.numpy as jnp
from jax import lax
from jax.experimental import pallas as pl
from jax.experimental.pallas import tpu as pltpu
from jax.experimental.pallas import tpu_sc as plsc

_NC = 2
_NS = 16
_NW = _NC * _NS
_C = 128
_KB = 16


def _sc_agg_body(x_hbm, srcr, dstr, z_row, agg_out,
                 idx_s, idx_d, rows0, rows1, acc_sh, semg0, semg1):
  ko_n = srcr.shape[1]
  cid = lax.axis_index("c")
  sid = lax.axis_index("s")
  wid = sid * _NC + cid
  nt = acc_sh.shape[0] // _NS
  pltpu.sync_copy(z_row, acc_sh.at[pl.ds(sid * nt, nt)])
  plsc.subcore_barrier()
  half = _KB // 2

  def outer(ko, carry):
    pltpu.sync_copy(srcr.at[wid, ko], idx_s)
    pltpu.sync_copy(dstr.at[wid, ko], idx_d)
    pltpu.async_copy(x_hbm.at[idx_s.at[0]], rows0, semg0)

    def inner(j2, c2):
      e = 2 * j2
      pltpu.async_copy(x_hbm.at[idx_s.at[e + 1]], rows1, semg1)
      pltpu.make_async_copy(x_hbm.at[idx_s.at[e]], rows0, semg0).wait()
      pltpu.sync_copy(rows0, acc_sh.at[idx_d.at[e]], add=True)

      @pl.when(j2 + 1 < half)
      def _():
        pltpu.async_copy(x_hbm.at[idx_s.at[e + 2]], rows0, semg0)

      pltpu.make_async_copy(x_hbm.at[idx_s.at[e + 1]], rows1, semg1).wait()
      pltpu.sync_copy(rows1, acc_sh.at[idx_d.at[e + 1]], add=True)
      return c2

    return lax.fori_loop(0, half, inner, carry)

  lax.fori_loop(0, ko_n, outer, 0)
  plsc.subcore_barrier()
  pltpu.sync_copy(acc_sh.at[pl.ds(sid * nt, nt)],
                  agg_out.at[cid, pl.ds(sid * nt, nt)])


def _make_sc_agg(nacc, D):
  mesh = plsc.VectorSubcoreMesh(core_axis_name="c", subcore_axis_name="s")
  return pl.kernel(
      _sc_agg_body,
      mesh=mesh,
      out_type=jax.ShapeDtypeStruct((_NC, nacc, D), jnp.float32),
      scratch_types=[
          pltpu.VMEM((_KB, _C), jnp.int32),
          pltpu.VMEM((_KB, _C), jnp.int32),
          pltpu.VMEM((_C, D), jnp.float32),
          pltpu.VMEM((_C, D), jnp.float32),
          pltpu.VMEM_SHARED((nacc, D), jnp.float32),
          pltpu.SemaphoreType.DMA,
          pltpu.SemaphoreType.DMA,
      ],
  )


def _sc_deg_body(dstr, ones_hb, z_row, deg_out, idx_d, rows, acc_sh, sems):
  ko_n = dstr.shape[1]
  cid = lax.axis_index("c")
  sid = lax.axis_index("s")
  wid = sid * _NC + cid
  nt = acc_sh.shape[0] // _NS
  pltpu.sync_copy(z_row, acc_sh.at[pl.ds(sid * nt, nt)])
  pltpu.sync_copy(ones_hb, rows)
  plsc.subcore_barrier()

  def outer(ko, carry):
    pltpu.sync_copy(dstr.at[wid, ko], idx_d)

    def fire(j, c2):
      pltpu.async_copy(rows, acc_sh.at[idx_d.at[j]], sems, add=True)
      return c2

    lax.fori_loop(0, _KB, fire, carry)

    def drain(j, c2):
      pltpu.make_async_copy(rows, acc_sh.at[idx_d.at[0]], sems).wait()
      return c2

    return lax.fori_loop(0, _KB, drain, carry)

  lax.fori_loop(0, ko_n, outer, 0)
  plsc.subcore_barrier()
  pltpu.sync_copy(acc_sh.at[pl.ds(sid * nt, nt)],
                  deg_out.at[cid, pl.ds(sid * nt, nt)])


def _make_sc_deg(nacc, D):
  mesh = plsc.VectorSubcoreMesh(core_axis_name="c", subcore_axis_name="s")
  return pl.kernel(
      _sc_deg_body,
      mesh=mesh,
      out_type=jax.ShapeDtypeStruct((_NC, nacc, D), jnp.float32),
      scratch_types=[
          pltpu.VMEM((_KB, _C), jnp.int32),
          pltpu.VMEM((_C, D), jnp.float32),
          pltpu.VMEM_SHARED((nacc, D), jnp.float32),
          pltpu.SemaphoreType.DMA,
      ],
  )


def _dotT(a, w):
  return lax.dot_general(a, w, (((1,), (1,)), ((), ())),
                         preferred_element_type=jnp.float32)


def _tc1_body(aggp, degp, x, w1l, b1l, w1r, h_out):
  agg = aggp[0] + aggp[1]
  deg = degp[0, :, 0:1] + degp[1, :, 0:1]
  a = agg / jnp.maximum(deg, 1.0)
  y = _dotT(a, w1l[...]) + b1l[...] + _dotT(x[...], w1r[...])
  h_out[...] = jnp.maximum(y, 0.0)


def _tc0_body(img, attr, rel, wi, bi, wa, ba, wr, br,
              im_out, at_out, re_out):
  im_out[...] = _dotT(img[...], wi[...]) + bi[...]
  at_out[...] = _dotT(attr[...], wa[...]) + ba[...]
  re_out[...] = _dotT(rel[...], wr[...]) + br[...]


def _tc2_body(aggp, degp, h, im_in, at_in, re_in,
              w2l, b2l, w2r, wg, bg, wf, bf,
              fused_out, h2_out):
  agg = aggp[0] + aggp[1]
  deg = degp[0, :, 0:1] + degp[1, :, 0:1]
  a = agg / jnp.maximum(deg, 1.0)
  h2 = _dotT(a, w2l[...]) + b2l[...] + _dotT(h[...], w2r[...])
  g = _dotT(h2, wg[...]) + bg[...]
  im = im_in[...]
  at = at_in[...]
  re = re_in[...]

  wfv = wf[...]
  b = bf[:, 0:1]
  lg = jnp.sum(g * wfv, axis=1, keepdims=True) + b
  li = jnp.sum(im * wfv, axis=1, keepdims=True) + b
  la = jnp.sum(at * wfv, axis=1, keepdims=True) + b
  lr = jnp.sum(re * wfv, axis=1, keepdims=True) + b
  m = jnp.maximum(jnp.maximum(lg, li), jnp.maximum(la, lr))
  eg = jnp.exp(lg - m)
  ei = jnp.exp(li - m)
  ea = jnp.exp(la - m)
  er = jnp.exp(lr - m)
  s = eg + ei + ea + er
  fused = (eg * g + ei * im + ea * at + er * re) / s
  nrm = jnp.sqrt(jnp.sum(fused * fused, axis=1, keepdims=True))
  fused_out[...] = fused / jnp.maximum(nrm, 1e-12)
  h2_out[...] = h2


def _row_spec(R, D):
  return pl.BlockSpec((R, D), lambda i: (i, 0))


def _full_spec(shape):
  nd = len(shape)
  return pl.BlockSpec(shape, lambda i: (0,) * nd)


def _tc0(img, attr, rel, *ws):
  N, D = img.shape
  R = 2000
  out = jax.ShapeDtypeStruct((N, D), jnp.float32)
  w_specs = [
      _full_spec((D, D)), _full_spec((1, D)),
      _full_spec((D, D)), _full_spec((1, D)),
      _full_spec((D, D)), _full_spec((1, D)),
  ]
  return pl.pallas_call(
      _tc0_body,
      grid=(N // R,),
      in_specs=[_row_spec(R, D)] * 3 + w_specs,
      out_specs=[_row_spec(R, D)] * 3,
      out_shape=[out, out, out],
  )(img, attr, rel, *ws)


def _tc1(aggp, degp, x, w1l, b1l, w1r):
  N, D = x.shape
  R = 2000
  return pl.pallas_call(
      _tc1_body,
      grid=(N // R,),
      in_specs=[
          pl.BlockSpec((_NC, R, D), lambda i: (0, i, 0)),
          pl.BlockSpec((_NC, R, D), lambda i: (0, i, 0)),
          _row_spec(R, D),
          _full_spec((D, D)),
          _full_spec((1, D)),
          _full_spec((D, D)),
      ],
      out_specs=_row_spec(R, D),
      out_shape=jax.ShapeDtypeStruct((N, D), jnp.float32),
  )(aggp, degp, x, w1l, b1l, w1r)


def _tc2(aggp, degp, h, im, at, re, *ws):
  N, D = h.shape
  R = 2000
  out = jax.ShapeDtypeStruct((N, D), jnp.float32)
  w_specs = [
      _full_spec((D, D)), _full_spec((1, D)), _full_spec((D, D)),
      _full_spec((D, D)), _full_spec((1, D)),
      _full_spec((1, D)), _full_spec((1, D)),
  ]
  return pl.pallas_call(
      _tc2_body,
      grid=(N // R,),
      in_specs=[
          pl.BlockSpec((_NC, R, D), lambda i: (0, i, 0)),
          pl.BlockSpec((_NC, R, D), lambda i: (0, i, 0)),
          _row_spec(R, D), _row_spec(R, D), _row_spec(R, D), _row_spec(R, D),
      ] + w_specs,
      out_specs=[_row_spec(R, D)] * 2,
      out_shape=[out, out],
  )(aggp, degp, h, im, at, re, *ws)


def kernel(x, edge_index, img_emb, attr_emb, rel_emb,
           W1l, b1l, W1r, W2l, b2l, W2r, Wgph, bgph, Wimg, bimg,
           Watt, batt, Wrel, brel, Wfus, bfus):
  N, D = x.shape
  E = edge_index.shape[1]

  ko_n = -(-E // (_NW * _KB * _C))
  pad = _NW * ko_n * _KB * _C - E
  src = edge_index[0].astype(jnp.int32)
  dst = edge_index[1].astype(jnp.int32)
  nacc = ((N + 96 + 8 * _NS - 1) // (8 * _NS)) * (8 * _NS)
  if pad:
    ar = jnp.arange(pad, dtype=jnp.int32)
    src = jnp.concatenate([src, (ar * 97) % N])
    dst = jnp.concatenate([dst, N + (ar % 96)])
  srcr = src.reshape(_NW, ko_n, _KB, _C)
  dstr = dst.reshape(_NW, ko_n, _KB, _C)

  nt = nacc // _NS
  z_row = jnp.zeros((nt, D), jnp.float32)

  sc_agg = _make_sc_agg(nacc, D)
  im, at, re = _tc0(img_emb, attr_emb, rel_emb,
                    Wimg, bimg.reshape(1, D),
                    Watt, batt.reshape(1, D),
                    Wrel, brel.reshape(1, D))
  ones_hb = jnp.ones((_C, D), jnp.float32)
  degp = _make_sc_deg(nacc, D)(dstr, ones_hb, z_row)
  agg1p = sc_agg(x, srcr, dstr, z_row)
  h = _tc1(agg1p, degp, x, W1l, b1l.reshape(1, D), W1r)
  agg2p = sc_agg(h, srcr, dstr, z_row)
  fused, h2 = _tc2(
      agg2p, degp, h, im, at, re,
      W2l, b2l.reshape(1, D), W2r,
      Wgph, bgph.reshape(1, D),
      Wfus, jnp.broadcast_to(bfus.reshape(1, 1), (1, D)))
  return fused, h2, im, at, re
